# baseline (speedup 1.0000x reference)
import functools

import jax
import jax.numpy as jnp
from jax import lax
from jax.experimental import pallas as pl
from jax.experimental.pallas import tpu as pltpu
from jax.experimental.pallas import tpu_sc as plsc

N = 10000
E = 320000
D_FEAT = 128
D_HID = 16

NPAD = 10240
NW = 32
SBLK = 1024
NSB = 10
EPW = NSB * SBLK
EPAD = NW * EPW
DUMMY = N
STRIPE = NPAD // 16

_mesh = plsc.VectorSubcoreMesh(core_axis_name="c", subcore_axis_name="s")
_sc_params = pltpu.CompilerParams(use_tc_tiling_on_sc=False)
_f32 = jnp.float32


def _zero_shared(z_hbm, shared, s):
    pltpu.sync_copy(z_hbm.at[pl.ds(s * STRIPE, STRIPE)],
                    shared.at[pl.ds(s * STRIPE, STRIPE)])


def _flush_shared(shared, out_hbm, c, s):
    pltpu.sync_copy(shared.at[pl.ds(s * STRIPE, STRIPE)],
                    out_hbm.at[c, pl.ds(s * STRIPE, STRIPE)])


@functools.partial(
    pl.kernel,
    out_type=jax.ShapeDtypeStruct((2, NPAD, D_HID), _f32),
    mesh=_mesh,
    scratch_types=[
        pltpu.VMEM((NSB, SBLK), jnp.int32),
        pltpu.VMEM((SBLK, D_HID), _f32),
        pltpu.VMEM_SHARED((NPAD, D_HID), _f32),
        pltpu.SemaphoreType.DMA,
    ],
    compiler_params=_sc_params,
)
def _deg_pass(dst_hbm, z_hbm, ones_hbm, out_hbm, dst_v, ones_v, shared, sem):
    c = lax.axis_index("c")
    s = lax.axis_index("s")
    w = c * 16 + s
    pltpu.sync_copy(dst_hbm.at[w], dst_v)
    pltpu.sync_copy(ones_hbm, ones_v)
    _zero_shared(z_hbm, shared, s)
    plsc.subcore_barrier()

    for i in range(NSB):
        pltpu.async_copy(ones_v, shared.at[dst_v.at[i]], sem, add=True)
    for i in range(NSB):
        pltpu.make_async_copy(ones_v, shared.at[dst_v.at[i]], sem).wait()
    plsc.subcore_barrier()
    _flush_shared(shared, out_hbm, c, s)


@functools.partial(
    pl.kernel,
    out_type=jax.ShapeDtypeStruct((2, NPAD, D_HID), _f32),
    mesh=_mesh,
    scratch_types=[
        pltpu.VMEM((NSB, SBLK), jnp.int32),
        pltpu.VMEM((NSB, SBLK), jnp.int32),
        [pltpu.VMEM((SBLK, D_HID), _f32) for _ in range(4)],
        [pltpu.SemaphoreType.DMA for _ in range(4)],
        [pltpu.SemaphoreType.DMA for _ in range(4)],
        pltpu.VMEM_SHARED((NPAD, D_HID), _f32),
    ],
    compiler_params=_sc_params,
)
def _edge_pass(g_hbm, src_hbm, dst_hbm, z_hbm, out_hbm,
               src_v, dst_v, bufs, gsems, ssems, shared):
    c = lax.axis_index("c")
    s = lax.axis_index("s")
    w = c * 16 + s
    pltpu.sync_copy(src_hbm.at[w], src_v)
    pltpu.sync_copy(dst_hbm.at[w], dst_v)
    _zero_shared(z_hbm, shared, s)
    plsc.subcore_barrier()

    def gather(i):
        return pltpu.async_copy(g_hbm.at[src_v.at[i]], bufs[i % 4], gsems[i % 4])

    def scatter(i):
        return pltpu.async_copy(bufs[i % 4], shared.at[dst_v.at[i]],
                                ssems[i % 4], add=True)

    for i in range(3):
        gather(i)
    for i in range(NSB):
        if i + 3 < NSB:
            if i >= 1:
                pltpu.make_async_copy(bufs[(i - 1) % 4],
                                      shared.at[dst_v.at[i - 1]],
                                      ssems[(i - 1) % 4]).wait()
            gather(i + 3)
        pltpu.make_async_copy(g_hbm.at[src_v.at[i]], bufs[i % 4],
                              gsems[i % 4]).wait()
        scatter(i)
    for i in range(NSB - 4, NSB):
        pltpu.make_async_copy(bufs[i % 4], shared.at[dst_v.at[i]],
                              ssems[i % 4]).wait()
    plsc.subcore_barrier()
    _flush_shared(shared, out_hbm, c, s)


def _tc_matmul1(x_p, W1):
    def body(x_ref, w_ref, h_ref):
        h_ref[...] = jnp.dot(x_ref[...], w_ref[...],
                             preferred_element_type=_f32)

    return pl.pallas_call(
        body,
        out_shape=jax.ShapeDtypeStruct((NPAD, D_HID), _f32),
    )(x_p, W1)


def _tc_layer1(h1, degp):
    def body(h_ref, d_ref, g_ref, dinv_ref):
        deg = d_ref[0] + d_ref[1] + 1.0
        dinv = lax.rsqrt(deg)
        g_ref[...] = h_ref[...] * dinv
        dinv_ref[...] = dinv

    return pl.pallas_call(
        body,
        out_shape=(jax.ShapeDtypeStruct((NPAD, D_HID), _f32),
                   jax.ShapeDtypeStruct((NPAD, D_HID), _f32)),
    )(h1, degp)


def _tc_layer2(sp1, g1, dinv, W2, b1):
    def body(sp_ref, g_ref, dinv_ref, w_ref, b_ref, g2_ref):
        s1 = sp_ref[0] + sp_ref[1] + g_ref[...]
        z = jnp.maximum(dinv_ref[...] * s1 + b_ref[...], 0.0)
        h2 = jnp.dot(z, w_ref[...], preferred_element_type=_f32)
        g2_ref[...] = h2 * dinv_ref[...]

    return pl.pallas_call(
        body,
        out_shape=jax.ShapeDtypeStruct((NPAD, D_HID), _f32),
    )(sp1, g1, dinv, W2, b1)


def _tc_out(sp2, g2, dinv, b2):
    def body(sp_ref, g_ref, dinv_ref, b_ref, out_ref):
        o = dinv_ref[...] * (sp_ref[0] + sp_ref[1] + g_ref[...]) + b_ref[...]
        m = jnp.max(o, axis=1, keepdims=True)
        e = o - m
        lse = jnp.log(jnp.sum(jnp.exp(e), axis=1, keepdims=True))
        out_ref[...] = e - lse

    return pl.pallas_call(
        body,
        out_shape=jax.ShapeDtypeStruct((NPAD, D_HID), _f32),
    )(sp2, g2, dinv, b2)


def kernel(x, edge_index, W1, b1, W2, b2):
    src = edge_index[0].astype(jnp.int32)
    dst = edge_index[1].astype(jnp.int32)
    pad = jnp.full((EPAD - E,), DUMMY, jnp.int32)
    srcp = jnp.concatenate([src, pad]).reshape(NW, NSB, SBLK)
    dstp = jnp.concatenate([dst, pad]).reshape(NW, NSB, SBLK)
    x_p = jnp.pad(x, ((0, NPAD - N), (0, 0)))
    z_t = jnp.zeros((NPAD, D_HID), _f32)
    ones_t = jnp.ones((SBLK, D_HID), _f32)

    degp = _deg_pass(dstp, z_t, ones_t)
    h1 = _tc_matmul1(x_p, W1)
    g1, dinv = _tc_layer1(h1, degp)
    sp1 = _edge_pass(g1, srcp, dstp, z_t)
    g2 = _tc_layer2(sp1, g1, dinv, W2, b1.reshape(1, D_HID))
    sp2 = _edge_pass(g2, srcp, dstp, z_t)
    out = _tc_out(sp2, g2, dinv, b2.reshape(1, D_HID))
    return out[:N]


# speedup vs baseline: 1.4090x; 1.4090x over previous
import functools

import jax
import jax.numpy as jnp
from jax import lax
from jax.experimental import pallas as pl
from jax.experimental.pallas import tpu as pltpu
from jax.experimental.pallas import tpu_sc as plsc

N = 10000
E = 320000
D_FEAT = 128
D_HID = 16

NPAD = 10240
NW = 32
SBLK = 1024
NSB = 10
EPW = NSB * SBLK
EPAD = NW * EPW
DUMMY = N
STRIPE = NPAD // 16

_mesh = plsc.VectorSubcoreMesh(core_axis_name="c", subcore_axis_name="s")
_sc_params = pltpu.CompilerParams(use_tc_tiling_on_sc=False)
_f32 = jnp.float32


def _zero_shared(z_hbm, shared, s):
    pltpu.sync_copy(z_hbm.at[pl.ds(s * STRIPE, STRIPE)],
                    shared.at[pl.ds(s * STRIPE, STRIPE)])


def _flush_shared(shared, out_hbm, c, s):
    pltpu.sync_copy(shared.at[pl.ds(s * STRIPE, STRIPE)],
                    out_hbm.at[c, pl.ds(s * STRIPE, STRIPE)])


@functools.partial(
    pl.kernel,
    out_type=jax.ShapeDtypeStruct((2, NPAD, D_HID), _f32),
    mesh=_mesh,
    scratch_types=[
        pltpu.VMEM((NSB, SBLK), jnp.int32),
        pltpu.VMEM((SBLK, D_HID), _f32),
        pltpu.VMEM_SHARED((NPAD, D_HID), _f32),
        pltpu.SemaphoreType.DMA,
    ],
    compiler_params=_sc_params,
)
def _deg_pass(dst_hbm, z_hbm, ones_hbm, out_hbm, dst_v, ones_v, shared, sem):
    c = lax.axis_index("c")
    s = lax.axis_index("s")
    w = c * 16 + s
    pltpu.sync_copy(dst_hbm.at[w], dst_v)
    pltpu.sync_copy(ones_hbm, ones_v)
    _zero_shared(z_hbm, shared, s)
    plsc.subcore_barrier()

    for i in range(NSB):
        pltpu.async_copy(ones_v, shared.at[dst_v.at[i]], sem, add=True)
    for i in range(NSB):
        pltpu.make_async_copy(ones_v, shared.at[dst_v.at[i]], sem).wait()
    plsc.subcore_barrier()
    _flush_shared(shared, out_hbm, c, s)


@functools.partial(
    pl.kernel,
    out_type=jax.ShapeDtypeStruct((2, NPAD, D_HID), _f32),
    mesh=_mesh,
    scratch_types=[
        pltpu.VMEM((NSB, SBLK), jnp.int32),
        pltpu.VMEM((NSB, SBLK), jnp.int32),
        [pltpu.VMEM((SBLK, D_HID), _f32) for _ in range(4)],
        [pltpu.SemaphoreType.DMA for _ in range(4)],
        [pltpu.SemaphoreType.DMA for _ in range(4)],
        pltpu.VMEM_SHARED((NPAD, D_HID), _f32),
        pltpu.VMEM_SHARED((NPAD, D_HID), _f32),
    ],
    compiler_params=_sc_params,
)
def _edge_pass(g_hbm, src_hbm, dst_hbm, z_hbm, out_hbm,
               src_v, dst_v, bufs, gsems, ssems, shared, shared_g):
    c = lax.axis_index("c")
    s = lax.axis_index("s")
    w = c * 16 + s
    pltpu.sync_copy(src_hbm.at[w], src_v)
    pltpu.sync_copy(dst_hbm.at[w], dst_v)
    pltpu.sync_copy(g_hbm.at[pl.ds(s * STRIPE, STRIPE)],
                    shared_g.at[pl.ds(s * STRIPE, STRIPE)])
    _zero_shared(z_hbm, shared, s)
    plsc.subcore_barrier()

    def gather(i):
        return pltpu.async_copy(shared_g.at[src_v.at[i]], bufs[i % 4],
                                gsems[i % 4])

    def scatter(i):
        return pltpu.async_copy(bufs[i % 4], shared.at[dst_v.at[i]],
                                ssems[i % 4], add=True)

    for i in range(3):
        gather(i)
    for i in range(NSB):
        if i + 3 < NSB:
            if i >= 1:
                pltpu.make_async_copy(bufs[(i - 1) % 4],
                                      shared.at[dst_v.at[i - 1]],
                                      ssems[(i - 1) % 4]).wait()
            gather(i + 3)
        pltpu.make_async_copy(shared_g.at[src_v.at[i]], bufs[i % 4],
                              gsems[i % 4]).wait()
        scatter(i)
    for i in range(NSB - 4, NSB):
        pltpu.make_async_copy(bufs[i % 4], shared.at[dst_v.at[i]],
                              ssems[i % 4]).wait()
    plsc.subcore_barrier()
    _flush_shared(shared, out_hbm, c, s)


def _tc_matmul1(x_p, W1):
    def body(x_ref, w_ref, h_ref):
        h_ref[...] = jnp.dot(x_ref[...], w_ref[...],
                             preferred_element_type=_f32)

    return pl.pallas_call(
        body,
        out_shape=jax.ShapeDtypeStruct((NPAD, D_HID), _f32),
    )(x_p, W1)


def _tc_layer1(h1, degp):
    def body(h_ref, d_ref, g_ref, dinv_ref):
        deg = d_ref[0] + d_ref[1] + 1.0
        dinv = lax.rsqrt(deg)
        g_ref[...] = h_ref[...] * dinv
        dinv_ref[...] = dinv

    return pl.pallas_call(
        body,
        out_shape=(jax.ShapeDtypeStruct((NPAD, D_HID), _f32),
                   jax.ShapeDtypeStruct((NPAD, D_HID), _f32)),
    )(h1, degp)


def _tc_layer2(sp1, g1, dinv, W2, b1):
    def body(sp_ref, g_ref, dinv_ref, w_ref, b_ref, g2_ref):
        s1 = sp_ref[0] + sp_ref[1] + g_ref[...]
        z = jnp.maximum(dinv_ref[...] * s1 + b_ref[...], 0.0)
        h2 = jnp.dot(z, w_ref[...], preferred_element_type=_f32)
        g2_ref[...] = h2 * dinv_ref[...]

    return pl.pallas_call(
        body,
        out_shape=jax.ShapeDtypeStruct((NPAD, D_HID), _f32),
    )(sp1, g1, dinv, W2, b1)


def _tc_out(sp2, g2, dinv, b2):
    def body(sp_ref, g_ref, dinv_ref, b_ref, out_ref):
        o = dinv_ref[...] * (sp_ref[0] + sp_ref[1] + g_ref[...]) + b_ref[...]
        m = jnp.max(o, axis=1, keepdims=True)
        e = o - m
        lse = jnp.log(jnp.sum(jnp.exp(e), axis=1, keepdims=True))
        out_ref[...] = e - lse

    return pl.pallas_call(
        body,
        out_shape=jax.ShapeDtypeStruct((NPAD, D_HID), _f32),
    )(sp2, g2, dinv, b2)


def kernel(x, edge_index, W1, b1, W2, b2):
    src = edge_index[0].astype(jnp.int32)
    dst = edge_index[1].astype(jnp.int32)
    pad = jnp.full((EPAD - E,), DUMMY, jnp.int32)
    srcp = jnp.concatenate([src, pad]).reshape(NW, NSB, SBLK)
    dstp = jnp.concatenate([dst, pad]).reshape(NW, NSB, SBLK)
    x_p = jnp.pad(x, ((0, NPAD - N), (0, 0)))
    z_t = jnp.zeros((NPAD, D_HID), _f32)
    ones_t = jnp.ones((SBLK, D_HID), _f32)

    degp = _deg_pass(dstp, z_t, ones_t)
    h1 = _tc_matmul1(x_p, W1)
    g1, dinv = _tc_layer1(h1, degp)
    sp1 = _edge_pass(g1, srcp, dstp, z_t)
    g2 = _tc_layer2(sp1, g1, dinv, W2, b1.reshape(1, D_HID))
    sp2 = _edge_pass(g2, srcp, dstp, z_t)
    out = _tc_out(sp2, g2, dinv, b2.reshape(1, D_HID))
    return out[:N]
